# baseline (speedup 1.0000x reference)
import functools

import jax
import jax.numpy as jnp
import numpy as np
from jax import lax
from jax.experimental import pallas as pl
from jax.experimental.pallas import tpu as pltpu
from jax.experimental.pallas import tpu_sc as plsc

B, S, D = 2, 2048, 768
E, K, P = 16, 2, 1536
T = B * S
TK = T * K
C = int(np.ceil(TK * 1.25 / E))
CP = C + 8
ECP = E * CP
TRASH = C
TOK_BLK = 256


def _route_body(x_ref, scale_ref, bias_ref, wg_ref, tril_ref,
                xn_ref, dst_ref, wv_ref, cnt_ref):
    i = pl.program_id(0)

    @pl.when(i == 0)
    def _():
        cnt_ref[...] = jnp.zeros_like(cnt_ref)

    xb = x_ref[...]
    mu = jnp.mean(xb, axis=-1, keepdims=True)
    xc = xb - mu
    var = jnp.mean(xc * xc, axis=-1, keepdims=True)
    xn = xc * jax.lax.rsqrt(var + 1e-5) * scale_ref[...] + bias_ref[...]
    xn_ref[...] = xn

    logits = jnp.dot(xn, wg_ref[...], preferred_element_type=jnp.float32,
                     precision=jax.lax.Precision.DEFAULT)
    eidx = jax.lax.broadcasted_iota(jnp.int32, (TOK_BLK, E), 1)
    m1 = jnp.max(logits, axis=-1, keepdims=True)
    i1 = jnp.min(jnp.where(logits >= m1, eidx, E), axis=-1, keepdims=True)
    l2 = jnp.where(eidx == i1, -jnp.inf, logits)
    m2 = jnp.max(l2, axis=-1, keepdims=True)
    i2 = jnp.min(jnp.where(l2 >= m2, eidx, E), axis=-1, keepdims=True)
    dexp = jnp.exp(m2 - m1)
    denom = 1.0 + dexp
    w1 = 1.0 / denom
    w2 = dexp / denom

    o1 = (eidx == i1).astype(jnp.float32)
    o2 = (eidx == i2).astype(jnp.float32)
    h = o1 + o2
    s = jnp.dot(tril_ref[...], h, preferred_element_type=jnp.float32) \
        + cnt_ref[...]
    cnt_ref[...] = cnt_ref[...] + jnp.sum(h, axis=0, keepdims=True)
    r1 = jnp.sum(s * o1, axis=-1, keepdims=True)
    r2 = jnp.sum(s * o2, axis=-1, keepdims=True)
    v1 = r1 < C
    v2 = r2 < C
    d1 = jnp.where(v1, i1 * CP + r1.astype(jnp.int32), TRASH)
    d2 = jnp.where(v2, i2 * CP + r2.astype(jnp.int32), TRASH)
    dst_ref[...] = jnp.concatenate([d1, d2], axis=1)
    w1m = jnp.where(v1, w1, 0.0)
    w2m = jnp.where(v2, w2, 0.0)
    wv_ref[...] = jnp.concatenate(
        [jnp.broadcast_to(w1m[:, None, :], (TOK_BLK, 1, 16)),
         jnp.broadcast_to(w2m[:, None, :], (TOK_BLK, 1, 16))], axis=1)


@functools.partial(jax.jit, static_argnames=("interpret",))
def _route(xf, ln_scale, ln_bias, wg, interpret=False):
    tril = jnp.tril(jnp.ones((TOK_BLK, TOK_BLK), jnp.float32), -1)
    grid = (T // TOK_BLK,)
    return pl.pallas_call(
        _route_body,
        grid=grid,
        in_specs=[
            pl.BlockSpec((TOK_BLK, D), lambda i: (i, 0)),
            pl.BlockSpec((1, D), lambda i: (0, 0)),
            pl.BlockSpec((1, D), lambda i: (0, 0)),
            pl.BlockSpec((D, E), lambda i: (0, 0)),
            pl.BlockSpec((TOK_BLK, TOK_BLK), lambda i: (0, 0)),
        ],
        out_specs=[
            pl.BlockSpec((TOK_BLK, D), lambda i: (i, 0)),
            pl.BlockSpec((TOK_BLK, 2), lambda i: (i, 0)),
            pl.BlockSpec((TOK_BLK, 2, 16), lambda i: (i, 0, 0)),
        ],
        out_shape=[
            jax.ShapeDtypeStruct((T, D), jnp.float32),
            jax.ShapeDtypeStruct((T, 2), jnp.int32),
            jax.ShapeDtypeStruct((T, 2, 16), jnp.float32),
        ],
        scratch_shapes=[pltpu.VMEM((1, E), jnp.float32)],
        compiler_params=pltpu.CompilerParams(
            dimension_semantics=("arbitrary",)),
        interpret=interpret,
    )(xf, ln_scale.reshape(1, D), ln_bias.reshape(1, D), wg, tril)


NC, NS, L = 2, 16, 16
NW = NC * NS
PAIRS_W = TK // NW
PCH = 64
TOK_W = T // NW
TCH = 32

_sc_mesh = plsc.VectorSubcoreMesh(core_axis_name="c", subcore_axis_name="s")


def _dispatch_body(xn_hbm, dst_hbm, disp_hbm, dstv, sidx, buf, sem):
    wid = lax.axis_index("s") * NC + lax.axis_index("c")

    def chunk(ci, carry):
        base = wid * PAIRS_W + ci * PCH
        pltpu.sync_copy(dst_hbm.at[pl.ds(base, PCH)], dstv)
        for g in range(PCH // L):
            pairv = base + g * L + lax.iota(jnp.int32, L)
            sidx[pl.ds(g * L, L)] = lax.shift_right_logical(pairv, 1)
        pltpu.async_copy(xn_hbm.at[sidx], buf, sem).wait()
        pltpu.async_copy(buf, disp_hbm.at[dstv], sem).wait()
        return carry

    lax.fori_loop(0, PAIRS_W // PCH, chunk, 0)


def _dispatch(xn, dstf):
    return pl.kernel(
        _dispatch_body,
        out_type=jax.ShapeDtypeStruct((ECP, D), jnp.float32),
        mesh=_sc_mesh,
        scratch_types=[
            pltpu.VMEM((PCH,), jnp.int32),
            pltpu.VMEM((PCH,), jnp.int32),
            pltpu.VMEM((PCH, D), jnp.float32),
            pltpu.SemaphoreType.DMA,
        ],
    )(xn, dstf)


def _cgather_body(dst_hbm, yw_hbm, g_hbm, dstv, buf, sem):
    wid = lax.axis_index("s") * NC + lax.axis_index("c")

    def chunk(ci, carry):
        base = wid * PAIRS_W + ci * PCH
        pltpu.sync_copy(dst_hbm.at[pl.ds(base, PCH)], dstv)
        pltpu.async_copy(yw_hbm.at[dstv], buf, sem).wait()
        pltpu.sync_copy(buf, g_hbm.at[pl.ds(base, PCH)])
        return carry

    lax.fori_loop(0, PAIRS_W // PCH, chunk, 0)


def _cgather(dstf, yw):
    return pl.kernel(
        _cgather_body,
        out_type=jax.ShapeDtypeStruct((TK, D), jnp.float32),
        mesh=_sc_mesh,
        scratch_types=[
            pltpu.VMEM((PCH,), jnp.int32),
            pltpu.VMEM((PCH, D), jnp.float32),
            pltpu.SemaphoreType.DMA,
        ],
    )(dstf, yw)


EBLK = 512


def _epi_body(x_ref, g_ref, wv_ref, o_ref):
    w = wv_ref[...]
    g = g_ref[...]
    o_ref[...] = (x_ref[...] + w[:, 0, 0:1] * g[:, 0, :]
                  + w[:, 1, 0:1] * g[:, 1, :])


@functools.partial(jax.jit, static_argnames=("interpret",))
def _epilogue(xf, g, wv, interpret=False):
    return pl.pallas_call(
        _epi_body,
        grid=(T // EBLK,),
        in_specs=[
            pl.BlockSpec((EBLK, D), lambda i: (i, 0)),
            pl.BlockSpec((EBLK, 2, D), lambda i: (i, 0, 0)),
            pl.BlockSpec((EBLK, 2, 16), lambda i: (i, 0, 0)),
        ],
        out_specs=pl.BlockSpec((EBLK, D), lambda i: (i, 0)),
        out_shape=jax.ShapeDtypeStruct((T, D), jnp.float32),
        compiler_params=pltpu.CompilerParams(
            dimension_semantics=("arbitrary",)),
        interpret=interpret,
    )(xf, g.reshape(T, 2, D), wv)


CB = CP


def _ffn_body(disp_ref, w1_ref, b1_ref, w2_ref, b2_ref, y_ref):
    xb = disp_ref[...].astype(jnp.bfloat16)
    a = jnp.dot(xb, w1_ref[0].astype(jnp.bfloat16),
                preferred_element_type=jnp.float32) + b1_ref[0]
    hh = 0.5 * a * (1.0 + jax.lax.erf(a * np.float32(1.0 / np.sqrt(2.0))))
    y_ref[...] = jnp.dot(hh.astype(jnp.bfloat16), w2_ref[0].astype(jnp.bfloat16),
                         preferred_element_type=jnp.float32) + b2_ref[0]


@functools.partial(jax.jit, static_argnames=("interpret",))
def _ffn(disp, W1, b1, W2, b2, interpret=False):
    return pl.pallas_call(
        _ffn_body,
        grid=(E,),
        in_specs=[
            pl.BlockSpec((CB, D), lambda e: (e, 0)),
            pl.BlockSpec((1, D, P), lambda e: (e, 0, 0)),
            pl.BlockSpec((1, 1, P), lambda e: (e, 0, 0)),
            pl.BlockSpec((1, P, D), lambda e: (e, 0, 0)),
            pl.BlockSpec((1, 1, D), lambda e: (e, 0, 0)),
        ],
        out_specs=pl.BlockSpec((CB, D), lambda e: (e, 0)),
        out_shape=jax.ShapeDtypeStruct((ECP, D), jnp.float32),
        compiler_params=pltpu.CompilerParams(
            dimension_semantics=("arbitrary",)),
        interpret=interpret,
    )(disp, W1, b1.reshape(E, 1, P), W2, b2.reshape(E, 1, D))


def kernel(x, ln_scale, ln_bias, Wg, W1, b1, W2, b2):
    xf = x.reshape(T, D)
    xn, dst, wv = _route(xf, ln_scale, ln_bias, Wg)
    dstf = dst.reshape(TK)
    disp = _dispatch(xn, dstf)
    yw = _ffn(disp, W1, b1, W2, b2)
    g = _cgather(dstf, yw)
    out = _epilogue(xf, g, wv)
    return out.reshape(B, S, D)


# speedup vs baseline: 1.2872x; 1.2872x over previous
import functools

import jax
import jax.numpy as jnp
import numpy as np
from jax import lax
from jax.experimental import pallas as pl
from jax.experimental.pallas import tpu as pltpu
from jax.experimental.pallas import tpu_sc as plsc

B, S, D = 2, 2048, 768
E, K, P = 16, 2, 1536
T = B * S
TK = T * K
C = int(np.ceil(TK * 1.25 / E))
CP = C + 8
ECP = E * CP
TRASH = C
TOK_BLK = 256


def _route_body(x_ref, scale_ref, bias_ref, wg_ref, tril_ref,
                xn_ref, dst_ref, wv_ref, cnt_ref):
    i = pl.program_id(0)

    @pl.when(i == 0)
    def _():
        cnt_ref[...] = jnp.zeros_like(cnt_ref)

    xb = x_ref[...]
    mu = jnp.mean(xb, axis=-1, keepdims=True)
    xc = xb - mu
    var = jnp.mean(xc * xc, axis=-1, keepdims=True)
    xn = xc * jax.lax.rsqrt(var + 1e-5) * scale_ref[...] + bias_ref[...]
    xn_ref[...] = xn

    logits = jnp.dot(xn, wg_ref[...], preferred_element_type=jnp.float32,
                     precision=jax.lax.Precision.DEFAULT)
    eidx = jax.lax.broadcasted_iota(jnp.int32, (TOK_BLK, E), 1)
    m1 = jnp.max(logits, axis=-1, keepdims=True)
    i1 = jnp.min(jnp.where(logits >= m1, eidx, E), axis=-1, keepdims=True)
    l2 = jnp.where(eidx == i1, -jnp.inf, logits)
    m2 = jnp.max(l2, axis=-1, keepdims=True)
    i2 = jnp.min(jnp.where(l2 >= m2, eidx, E), axis=-1, keepdims=True)
    dexp = jnp.exp(m2 - m1)
    denom = 1.0 + dexp
    w1 = 1.0 / denom
    w2 = dexp / denom

    o1 = (eidx == i1).astype(jnp.float32)
    o2 = (eidx == i2).astype(jnp.float32)
    h = o1 + o2
    s = jnp.dot(tril_ref[...], h, preferred_element_type=jnp.float32) \
        + cnt_ref[...]
    cnt_ref[...] = cnt_ref[...] + jnp.sum(h, axis=0, keepdims=True)
    r1 = jnp.sum(s * o1, axis=-1, keepdims=True)
    r2 = jnp.sum(s * o2, axis=-1, keepdims=True)
    v1 = r1 < C
    v2 = r2 < C
    d1 = jnp.where(v1, i1 * CP + r1.astype(jnp.int32), TRASH)
    d2 = jnp.where(v2, i2 * CP + r2.astype(jnp.int32), TRASH)
    dst_ref[...] = jnp.concatenate([d1, d2], axis=1)
    w1m = jnp.where(v1, w1, 0.0)
    w2m = jnp.where(v2, w2, 0.0)
    wv_ref[...] = jnp.concatenate(
        [jnp.broadcast_to(w1m[:, None, :], (TOK_BLK, 1, 16)),
         jnp.broadcast_to(w2m[:, None, :], (TOK_BLK, 1, 16))], axis=1)


@functools.partial(jax.jit, static_argnames=("interpret",))
def _route(xf, ln_scale, ln_bias, wg, interpret=False):
    tril = jnp.tril(jnp.ones((TOK_BLK, TOK_BLK), jnp.float32), -1)
    grid = (T // TOK_BLK,)
    return pl.pallas_call(
        _route_body,
        grid=grid,
        in_specs=[
            pl.BlockSpec((TOK_BLK, D), lambda i: (i, 0)),
            pl.BlockSpec((1, D), lambda i: (0, 0)),
            pl.BlockSpec((1, D), lambda i: (0, 0)),
            pl.BlockSpec((D, E), lambda i: (0, 0)),
            pl.BlockSpec((TOK_BLK, TOK_BLK), lambda i: (0, 0)),
        ],
        out_specs=[
            pl.BlockSpec((TOK_BLK, D), lambda i: (i, 0)),
            pl.BlockSpec((TOK_BLK, 2), lambda i: (i, 0)),
            pl.BlockSpec((TOK_BLK, 2, 16), lambda i: (i, 0, 0)),
        ],
        out_shape=[
            jax.ShapeDtypeStruct((T, D), jnp.float32),
            jax.ShapeDtypeStruct((T, 2), jnp.int32),
            jax.ShapeDtypeStruct((T, 2, 16), jnp.float32),
        ],
        scratch_shapes=[pltpu.VMEM((1, E), jnp.float32)],
        compiler_params=pltpu.CompilerParams(
            dimension_semantics=("arbitrary",)),
        interpret=interpret,
    )(xf, ln_scale.reshape(1, D), ln_bias.reshape(1, D), wg, tril)


NC, NS, L = 2, 16, 16
NW = NC * NS
PAIRS_W = TK // NW
PCH = 64
TOK_W = T // NW
TCH = 16

_sc_mesh = plsc.VectorSubcoreMesh(core_axis_name="c", subcore_axis_name="s")


NCH_D = PAIRS_W // PCH


def _dispatch_body(xn_hbm, dst2_hbm, disp_hbm, dstv, sidx, buf0, buf1,
                   gsem, ssem):
    wid = lax.axis_index("s") * NC + lax.axis_index("c")
    bufs = (buf0, buf1)
    pltpu.sync_copy(dst2_hbm.at[pl.ds(wid * NCH_D, NCH_D)], dstv)
    for g in range(PAIRS_W // L):
        pairv = wid * PAIRS_W + g * L + lax.iota(jnp.int32, L)
        sidx[g // (PCH // L), pl.ds((g % (PCH // L)) * L, L)] = (
            lax.shift_right_logical(pairv, 1))
    gsems = gsem
    ssems = ssem
    pltpu.async_copy(xn_hbm.at[sidx.at[0]], bufs[0], gsems[0])
    for ci in range(NCH_D):
        p = ci % 2
        q = (ci + 1) % 2
        if ci >= 1:
            pltpu.make_async_copy(bufs[q], disp_hbm.at[dstv.at[ci - 1]],
                                  ssems[q]).wait()
        if ci + 1 < NCH_D:
            pltpu.async_copy(xn_hbm.at[sidx.at[ci + 1]], bufs[q], gsems[q])
        pltpu.make_async_copy(xn_hbm.at[sidx.at[ci]], bufs[p], gsems[p]).wait()
        pltpu.async_copy(bufs[p], disp_hbm.at[dstv.at[ci]], ssems[p])
    pltpu.make_async_copy(bufs[(NCH_D - 1) % 2],
                          disp_hbm.at[dstv.at[NCH_D - 1]],
                          ssems[(NCH_D - 1) % 2]).wait()


def _dispatch(xn, dstf):
    return pl.kernel(
        _dispatch_body,
        out_type=jax.ShapeDtypeStruct((ECP, D), jnp.float32),
        mesh=_sc_mesh,
        scratch_types=[
            pltpu.VMEM((NCH_D, PCH), jnp.int32),
            pltpu.VMEM((NCH_D, PCH), jnp.int32),
            pltpu.VMEM((PCH, D), jnp.float32),
            pltpu.VMEM((PCH, D), jnp.float32),
            (pltpu.SemaphoreType.DMA, pltpu.SemaphoreType.DMA),
            (pltpu.SemaphoreType.DMA, pltpu.SemaphoreType.DMA),
        ],
    )(xn, dstf.reshape(NW * NCH_D, PCH))


NCH_C = TOK_W // TCH


def _combine_body(x_hbm, dst2_hbm, wb_hbm, yw_hbm, out_hbm,
                  dstv, wbuf, xb0, xb1, gb0, gb1, gsem, xsem, osem):
    wid = lax.axis_index("s") * NC + lax.axis_index("c")
    xbs = (xb0, xb1)
    gbs = (gb0, gb1)
    pltpu.sync_copy(dst2_hbm.at[pl.ds(wid * NCH_C, NCH_C)], dstv)
    pltpu.sync_copy(wb_hbm.at[pl.ds(wid * 2 * TOK_W, 2 * TOK_W)], wbuf)
    pltpu.async_copy(yw_hbm.at[dstv.at[0]], gbs[0], gsem[0])
    pltpu.async_copy(x_hbm.at[pl.ds(wid * TOK_W, TCH)], xbs[0], xsem[0])
    for ci in range(NCH_C):
        p = ci % 2
        q = (ci + 1) % 2
        if ci >= 1:
            pltpu.make_async_copy(
                xbs[q], out_hbm.at[pl.ds(wid * TOK_W + (ci - 1) * TCH, TCH)],
                osem[q]).wait()
        if ci + 1 < NCH_C:
            pltpu.async_copy(yw_hbm.at[dstv.at[ci + 1]], gbs[q], gsem[q])
            pltpu.async_copy(
                x_hbm.at[pl.ds(wid * TOK_W + (ci + 1) * TCH, TCH)],
                xbs[q], xsem[q])
        pltpu.make_async_copy(yw_hbm.at[dstv.at[ci]], gbs[p], gsem[p]).wait()
        pltpu.make_async_copy(x_hbm.at[pl.ds(0, TCH)], xbs[p], xsem[p]).wait()

        xb = xbs[p]
        gb = gbs[p]

        def tok_row(t, carry):
            w1v = wbuf[ci * 2 * TCH + 2 * t, :]
            w2v = wbuf[ci * 2 * TCH + 2 * t + 1, :]
            for c in range(D // L):
                sl = pl.ds(c * L, L)
                xb[t, sl] = (xb[t, sl] + w1v * gb[2 * t, sl]
                             + w2v * gb[2 * t + 1, sl])
            return carry

        lax.fori_loop(0, TCH, tok_row, 0)
        pltpu.async_copy(xb, out_hbm.at[pl.ds(wid * TOK_W + ci * TCH, TCH)],
                         osem[p])
    pltpu.make_async_copy(
        xbs[(NCH_C - 1) % 2],
        out_hbm.at[pl.ds(wid * TOK_W + (NCH_C - 1) * TCH, TCH)],
        osem[(NCH_C - 1) % 2]).wait()


def _combine(xf, dstf, wb, yw):
    return pl.kernel(
        _combine_body,
        out_type=jax.ShapeDtypeStruct((T, D), jnp.float32),
        mesh=_sc_mesh,
        scratch_types=[
            pltpu.VMEM((NCH_C, 2 * TCH), jnp.int32),
            pltpu.VMEM((2 * TOK_W, 16), jnp.float32),
            pltpu.VMEM((TCH, D), jnp.float32),
            pltpu.VMEM((TCH, D), jnp.float32),
            pltpu.VMEM((2 * TCH, D), jnp.float32),
            pltpu.VMEM((2 * TCH, D), jnp.float32),
            (pltpu.SemaphoreType.DMA, pltpu.SemaphoreType.DMA),
            (pltpu.SemaphoreType.DMA, pltpu.SemaphoreType.DMA),
            (pltpu.SemaphoreType.DMA, pltpu.SemaphoreType.DMA),
        ],
    )(xf, dstf.reshape(NW * NCH_C, 2 * TCH), wb, yw)


CB = CP


def _ffn_body(disp_ref, w1_ref, b1_ref, w2_ref, b2_ref, y_ref):
    xb = disp_ref[...].astype(jnp.bfloat16)
    a = jnp.dot(xb, w1_ref[0].astype(jnp.bfloat16),
                preferred_element_type=jnp.float32) + b1_ref[0]
    hh = 0.5 * a * (1.0 + jax.lax.erf(a * np.float32(1.0 / np.sqrt(2.0))))
    y_ref[...] = jnp.dot(hh.astype(jnp.bfloat16), w2_ref[0].astype(jnp.bfloat16),
                         preferred_element_type=jnp.float32) + b2_ref[0]


@functools.partial(jax.jit, static_argnames=("interpret",))
def _ffn(disp, W1, b1, W2, b2, interpret=False):
    return pl.pallas_call(
        _ffn_body,
        grid=(E,),
        in_specs=[
            pl.BlockSpec((CB, D), lambda e: (e, 0)),
            pl.BlockSpec((1, D, P), lambda e: (e, 0, 0)),
            pl.BlockSpec((1, 1, P), lambda e: (e, 0, 0)),
            pl.BlockSpec((1, P, D), lambda e: (e, 0, 0)),
            pl.BlockSpec((1, 1, D), lambda e: (e, 0, 0)),
        ],
        out_specs=pl.BlockSpec((CB, D), lambda e: (e, 0)),
        out_shape=jax.ShapeDtypeStruct((ECP, D), jnp.float32),
        compiler_params=pltpu.CompilerParams(
            dimension_semantics=("arbitrary",)),
        interpret=interpret,
    )(disp, W1, b1.reshape(E, 1, P), W2, b2.reshape(E, 1, D))


def kernel(x, ln_scale, ln_bias, Wg, W1, b1, W2, b2):
    xf = x.reshape(T, D)
    xn, dst, wv = _route(xf, ln_scale, ln_bias, Wg)
    dstf = dst.reshape(TK)
    wb = wv.reshape(TK, 16)
    disp = _dispatch(xn, dstf)
    yw = _ffn(disp, W1, b1, W2, b2)
    out = _combine(xf, dstf, wb, yw)
    return out.reshape(B, S, D)


# speedup vs baseline: 1.3142x; 1.0210x over previous
import functools

import jax
import jax.numpy as jnp
import numpy as np
from jax import lax
from jax.experimental import pallas as pl
from jax.experimental.pallas import tpu as pltpu
from jax.experimental.pallas import tpu_sc as plsc

B, S, D = 2, 2048, 768
E, K, P = 16, 2, 1536
T = B * S
TK = T * K
C = int(np.ceil(TK * 1.25 / E))
CP = C + 8
ECP = E * CP
TRASH = C
TOK_BLK = 256


def _route_body(x_ref, scale_ref, bias_ref, wg_ref, tril_ref,
                xn_ref, dst_ref, wv_ref, cnt_ref):
    i = pl.program_id(0)

    @pl.when(i == 0)
    def _():
        cnt_ref[...] = jnp.zeros_like(cnt_ref)

    xb = x_ref[...]
    mu = jnp.mean(xb, axis=-1, keepdims=True)
    xc = xb - mu
    var = jnp.mean(xc * xc, axis=-1, keepdims=True)
    xn = xc * jax.lax.rsqrt(var + 1e-5) * scale_ref[...] + bias_ref[...]
    xn_ref[...] = xn

    logits = jnp.dot(xn, wg_ref[...], preferred_element_type=jnp.float32,
                     precision=jax.lax.Precision.DEFAULT)
    eidx = jax.lax.broadcasted_iota(jnp.int32, (TOK_BLK, E), 1)
    m1 = jnp.max(logits, axis=-1, keepdims=True)
    i1 = jnp.min(jnp.where(logits >= m1, eidx, E), axis=-1, keepdims=True)
    l2 = jnp.where(eidx == i1, -jnp.inf, logits)
    m2 = jnp.max(l2, axis=-1, keepdims=True)
    i2 = jnp.min(jnp.where(l2 >= m2, eidx, E), axis=-1, keepdims=True)
    dexp = jnp.exp(m2 - m1)
    denom = 1.0 + dexp
    w1 = 1.0 / denom
    w2 = dexp / denom

    o1 = (eidx == i1).astype(jnp.float32)
    o2 = (eidx == i2).astype(jnp.float32)
    h = o1 + o2
    s = jnp.dot(tril_ref[...], h, preferred_element_type=jnp.float32) \
        + cnt_ref[...]
    cnt_ref[...] = cnt_ref[...] + jnp.sum(h, axis=0, keepdims=True)
    r1 = jnp.sum(s * o1, axis=-1, keepdims=True)
    r2 = jnp.sum(s * o2, axis=-1, keepdims=True)
    v1 = r1 < C
    v2 = r2 < C
    d1 = jnp.where(v1, i1 * CP + r1.astype(jnp.int32), TRASH)
    d2 = jnp.where(v2, i2 * CP + r2.astype(jnp.int32), TRASH)
    dst_ref[...] = jnp.concatenate([d1, d2], axis=1)
    w1m = jnp.where(v1, w1, 0.0)
    w2m = jnp.where(v2, w2, 0.0)
    wv_ref[...] = jnp.concatenate(
        [jnp.broadcast_to(w1m[:, None, :], (TOK_BLK, 1, 128)),
         jnp.broadcast_to(w2m[:, None, :], (TOK_BLK, 1, 128))], axis=1)


@functools.partial(jax.jit, static_argnames=("interpret",))
def _route(xf, ln_scale, ln_bias, wg, interpret=False):
    tril = jnp.tril(jnp.ones((TOK_BLK, TOK_BLK), jnp.float32), -1)
    grid = (T // TOK_BLK,)
    return pl.pallas_call(
        _route_body,
        grid=grid,
        in_specs=[
            pl.BlockSpec((TOK_BLK, D), lambda i: (i, 0)),
            pl.BlockSpec((1, D), lambda i: (0, 0)),
            pl.BlockSpec((1, D), lambda i: (0, 0)),
            pl.BlockSpec((D, E), lambda i: (0, 0)),
            pl.BlockSpec((TOK_BLK, TOK_BLK), lambda i: (0, 0)),
        ],
        out_specs=[
            pl.BlockSpec((TOK_BLK, D), lambda i: (i, 0)),
            pl.BlockSpec((TOK_BLK, 2), lambda i: (i, 0)),
            pl.BlockSpec((TOK_BLK, 2, 128), lambda i: (i, 0, 0)),
        ],
        out_shape=[
            jax.ShapeDtypeStruct((T, D), jnp.float32),
            jax.ShapeDtypeStruct((T, 2), jnp.int32),
            jax.ShapeDtypeStruct((T, 2, 128), jnp.float32),
        ],
        scratch_shapes=[pltpu.VMEM((1, E), jnp.float32)],
        compiler_params=pltpu.CompilerParams(
            dimension_semantics=("arbitrary",)),
        interpret=interpret,
    )(xf, ln_scale.reshape(1, D), ln_bias.reshape(1, D), wg, tril)


NC, NS, L = 2, 16, 16
NW = NC * NS
PAIRS_W = TK // NW
PCH = 64
TOK_W = T // NW
TCH = 16

_sc_mesh = plsc.VectorSubcoreMesh(core_axis_name="c", subcore_axis_name="s")


NCH_D = PAIRS_W // PCH


def _dispatch_body(xn_hbm, dst2_hbm, wb_hbm, disp_hbm, wrow_hbm,
                   dstv, sidx, buf0, buf1, wb0, wb1, gsem, ssem, wsem):
    wid = lax.axis_index("s") * NC + lax.axis_index("c")
    bufs = (buf0, buf1)
    wbs = (wb0, wb1)
    pltpu.sync_copy(dst2_hbm.at[pl.ds(wid * NCH_D, NCH_D)], dstv)
    for g in range(PAIRS_W // L):
        pairv = wid * PAIRS_W + g * L + lax.iota(jnp.int32, L)
        sidx[g // (PCH // L), pl.ds((g % (PCH // L)) * L, L)] = (
            lax.shift_right_logical(pairv, 1))
    pltpu.async_copy(xn_hbm.at[sidx.at[0]], bufs[0], gsem[0])
    for ci in range(NCH_D):
        p = ci % 2
        q = (ci + 1) % 2
        if ci >= 1:
            pltpu.make_async_copy(bufs[q], disp_hbm.at[dstv.at[ci - 1]],
                                  ssem[q]).wait()
            pltpu.make_async_copy(wbs[q], wrow_hbm.at[dstv.at[ci - 1]],
                                  wsem[q]).wait()
        if ci + 1 < NCH_D:
            pltpu.async_copy(xn_hbm.at[sidx.at[ci + 1]], bufs[q], gsem[q])
        pltpu.sync_copy(wb_hbm.at[pl.ds(wid * PAIRS_W + ci * PCH, PCH)],
                        wbs[p])
        pltpu.make_async_copy(xn_hbm.at[sidx.at[ci]], bufs[p], gsem[p]).wait()
        pltpu.async_copy(bufs[p], disp_hbm.at[dstv.at[ci]], ssem[p])
        pltpu.async_copy(wbs[p], wrow_hbm.at[dstv.at[ci]], wsem[p])
    pltpu.make_async_copy(bufs[(NCH_D - 1) % 2],
                          disp_hbm.at[dstv.at[NCH_D - 1]],
                          ssem[(NCH_D - 1) % 2]).wait()
    pltpu.make_async_copy(wbs[(NCH_D - 1) % 2],
                          wrow_hbm.at[dstv.at[NCH_D - 1]],
                          wsem[(NCH_D - 1) % 2]).wait()


def _dispatch(xn, dstf, wb):
    return pl.kernel(
        _dispatch_body,
        out_type=[jax.ShapeDtypeStruct((ECP, D), jnp.float32),
                  jax.ShapeDtypeStruct((ECP, 128), jnp.float32)],
        mesh=_sc_mesh,
        scratch_types=[
            pltpu.VMEM((NCH_D, PCH), jnp.int32),
            pltpu.VMEM((NCH_D, PCH), jnp.int32),
            pltpu.VMEM((PCH, D), jnp.float32),
            pltpu.VMEM((PCH, D), jnp.float32),
            pltpu.VMEM((PCH, 128), jnp.float32),
            pltpu.VMEM((PCH, 128), jnp.float32),
            (pltpu.SemaphoreType.DMA, pltpu.SemaphoreType.DMA),
            (pltpu.SemaphoreType.DMA, pltpu.SemaphoreType.DMA),
            (pltpu.SemaphoreType.DMA, pltpu.SemaphoreType.DMA),
        ],
    )(xn, dstf.reshape(NW * NCH_D, PCH), wb)


NCH_C = TOK_W // TCH


def _combine_body(x_hbm, dst2_hbm, yw_hbm, out_hbm,
                  dstv, xb0, xb1, gb0, gb1, gsem, xsem, osem):
    wid = lax.axis_index("s") * NC + lax.axis_index("c")
    xbs = (xb0, xb1)
    gbs = (gb0, gb1)
    pltpu.sync_copy(dst2_hbm.at[pl.ds(wid * NCH_C, NCH_C)], dstv)
    pltpu.async_copy(yw_hbm.at[dstv.at[0]], gbs[0], gsem[0])
    pltpu.async_copy(x_hbm.at[pl.ds(wid * TOK_W, TCH)], xbs[0], xsem[0])
    for ci in range(NCH_C):
        p = ci % 2
        q = (ci + 1) % 2
        if ci >= 1:
            pltpu.make_async_copy(
                xbs[q], out_hbm.at[pl.ds(wid * TOK_W + (ci - 1) * TCH, TCH)],
                osem[q]).wait()
        if ci + 1 < NCH_C:
            pltpu.async_copy(yw_hbm.at[dstv.at[ci + 1]], gbs[q], gsem[q])
            pltpu.async_copy(
                x_hbm.at[pl.ds(wid * TOK_W + (ci + 1) * TCH, TCH)],
                xbs[q], xsem[q])
        pltpu.make_async_copy(yw_hbm.at[dstv.at[ci]], gbs[p], gsem[p]).wait()
        pltpu.make_async_copy(x_hbm.at[pl.ds(0, TCH)], xbs[p], xsem[p]).wait()

        xb = xbs[p]
        gb = gbs[p]

        def tok_row(t, carry):
            for c in range(D // L):
                sl = pl.ds(c * L, L)
                xb[t, sl] = xb[t, sl] + gb[2 * t, sl] + gb[2 * t + 1, sl]
            return carry

        lax.fori_loop(0, TCH, tok_row, 0)
        pltpu.async_copy(xb, out_hbm.at[pl.ds(wid * TOK_W + ci * TCH, TCH)],
                         osem[p])
    pltpu.make_async_copy(
        xbs[(NCH_C - 1) % 2],
        out_hbm.at[pl.ds(wid * TOK_W + (NCH_C - 1) * TCH, TCH)],
        osem[(NCH_C - 1) % 2]).wait()


def _combine(xf, dstf, yw):
    return pl.kernel(
        _combine_body,
        out_type=jax.ShapeDtypeStruct((T, D), jnp.float32),
        mesh=_sc_mesh,
        scratch_types=[
            pltpu.VMEM((NCH_C, 2 * TCH), jnp.int32),
            pltpu.VMEM((TCH, D), jnp.float32),
            pltpu.VMEM((TCH, D), jnp.float32),
            pltpu.VMEM((2 * TCH, D), jnp.float32),
            pltpu.VMEM((2 * TCH, D), jnp.float32),
            (pltpu.SemaphoreType.DMA, pltpu.SemaphoreType.DMA),
            (pltpu.SemaphoreType.DMA, pltpu.SemaphoreType.DMA),
            (pltpu.SemaphoreType.DMA, pltpu.SemaphoreType.DMA),
        ],
    )(xf, dstf.reshape(NW * NCH_C, 2 * TCH), yw)


CB = CP


def _ffn_body(disp_ref, w1_ref, b1_ref, w2_ref, b2_ref, wr_ref, y_ref):
    xb = disp_ref[...].astype(jnp.bfloat16)
    a = jnp.dot(xb, w1_ref[0].astype(jnp.bfloat16),
                preferred_element_type=jnp.float32) + b1_ref[0]
    hh = 0.5 * a * (1.0 + jax.lax.erf(a * np.float32(1.0 / np.sqrt(2.0))))
    y = jnp.dot(hh.astype(jnp.bfloat16), w2_ref[0].astype(jnp.bfloat16),
                preferred_element_type=jnp.float32) + b2_ref[0]
    y_ref[...] = y * wr_ref[...][:, 0:1]


@functools.partial(jax.jit, static_argnames=("interpret",))
def _ffn(disp, W1, b1, W2, b2, wrow, interpret=False):
    return pl.pallas_call(
        _ffn_body,
        grid=(E,),
        in_specs=[
            pl.BlockSpec((CB, D), lambda e: (e, 0)),
            pl.BlockSpec((1, D, P), lambda e: (e, 0, 0)),
            pl.BlockSpec((1, 1, P), lambda e: (e, 0, 0)),
            pl.BlockSpec((1, P, D), lambda e: (e, 0, 0)),
            pl.BlockSpec((1, 1, D), lambda e: (e, 0, 0)),
            pl.BlockSpec((CB, 128), lambda e: (e, 0)),
        ],
        out_specs=pl.BlockSpec((CB, D), lambda e: (e, 0)),
        out_shape=jax.ShapeDtypeStruct((ECP, D), jnp.float32),
        compiler_params=pltpu.CompilerParams(
            dimension_semantics=("arbitrary",)),
        interpret=interpret,
    )(disp, W1, b1.reshape(E, 1, P), W2, b2.reshape(E, 1, D), wrow)


def kernel(x, ln_scale, ln_bias, Wg, W1, b1, W2, b2):
    xf = x.reshape(T, D)
    xn, dst, wv = _route(xf, ln_scale, ln_bias, Wg)
    dstf = dst.reshape(TK)
    wb = wv.reshape(TK, 128)
    disp, wrow = _dispatch(xn, dstf, wb)
    yw = _ffn(disp, W1, b1, W2, b2, wrow)
    out = _combine(xf, dstf, yw)
    return out.reshape(B, S, D)


# speedup vs baseline: 1.3438x; 1.0225x over previous
import functools

import jax
import jax.numpy as jnp
import numpy as np
from jax import lax
from jax.experimental import pallas as pl
from jax.experimental.pallas import tpu as pltpu
from jax.experimental.pallas import tpu_sc as plsc

B, S, D = 2, 2048, 768
E, K, P = 16, 2, 1536
T = B * S
TK = T * K
C = int(np.ceil(TK * 1.25 / E))
CP = C + 8
ECP = E * CP
TRASH = C
TOK_BLK = 512


def _route_body(x_ref, scale_ref, bias_ref, wg_ref, tril_ref,
                xn_ref, dst_ref, wv_ref, cnt_ref):
    i = pl.program_id(0)

    @pl.when(i == 0)
    def _():
        cnt_ref[...] = jnp.zeros_like(cnt_ref)

    xb = x_ref[...]
    mu = jnp.mean(xb, axis=-1, keepdims=True)
    xc = xb - mu
    var = jnp.mean(xc * xc, axis=-1, keepdims=True)
    xn = xc * jax.lax.rsqrt(var + 1e-5) * scale_ref[...] + bias_ref[...]
    xn_ref[...] = xn

    logits = jnp.dot(xn, wg_ref[...], preferred_element_type=jnp.float32,
                     precision=jax.lax.Precision.DEFAULT)
    eidx = jax.lax.broadcasted_iota(jnp.int32, (TOK_BLK, E), 1)
    m1 = jnp.max(logits, axis=-1, keepdims=True)
    i1 = jnp.min(jnp.where(logits >= m1, eidx, E), axis=-1, keepdims=True)
    l2 = jnp.where(eidx == i1, -jnp.inf, logits)
    m2 = jnp.max(l2, axis=-1, keepdims=True)
    i2 = jnp.min(jnp.where(l2 >= m2, eidx, E), axis=-1, keepdims=True)
    dexp = jnp.exp(m2 - m1)
    denom = 1.0 + dexp
    w1 = 1.0 / denom
    w2 = dexp / denom

    o1 = (eidx == i1).astype(jnp.float32)
    o2 = (eidx == i2).astype(jnp.float32)
    h = o1 + o2
    s = jnp.dot(tril_ref[...], h, preferred_element_type=jnp.float32) \
        + cnt_ref[...]
    cnt_ref[...] = cnt_ref[...] + jnp.sum(h, axis=0, keepdims=True)
    r1 = jnp.sum(s * o1, axis=-1, keepdims=True)
    r2 = jnp.sum(s * o2, axis=-1, keepdims=True)
    v1 = r1 < C
    v2 = r2 < C
    d1 = jnp.where(v1, i1 * CP + r1.astype(jnp.int32), TRASH)
    d2 = jnp.where(v2, i2 * CP + r2.astype(jnp.int32), TRASH)
    dst_ref[...] = jnp.concatenate([d1, d2], axis=1)
    w1m = jnp.where(v1, w1, 0.0)
    w2m = jnp.where(v2, w2, 0.0)
    wv_ref[...] = jnp.concatenate(
        [jnp.broadcast_to(w1m[:, None, :], (TOK_BLK, 1, 128)),
         jnp.broadcast_to(w2m[:, None, :], (TOK_BLK, 1, 128))], axis=1)


@functools.partial(jax.jit, static_argnames=("interpret",))
def _route(xf, ln_scale, ln_bias, wg, interpret=False):
    tril = jnp.tril(jnp.ones((TOK_BLK, TOK_BLK), jnp.float32), -1)
    grid = (T // TOK_BLK,)
    return pl.pallas_call(
        _route_body,
        grid=grid,
        in_specs=[
            pl.BlockSpec((TOK_BLK, D), lambda i: (i, 0)),
            pl.BlockSpec((1, D), lambda i: (0, 0)),
            pl.BlockSpec((1, D), lambda i: (0, 0)),
            pl.BlockSpec((D, E), lambda i: (0, 0)),
            pl.BlockSpec((TOK_BLK, TOK_BLK), lambda i: (0, 0)),
        ],
        out_specs=[
            pl.BlockSpec((TOK_BLK, D), lambda i: (i, 0)),
            pl.BlockSpec((TOK_BLK, 2), lambda i: (i, 0)),
            pl.BlockSpec((TOK_BLK, 2, 128), lambda i: (i, 0, 0)),
        ],
        out_shape=[
            jax.ShapeDtypeStruct((T, D), jnp.float32),
            jax.ShapeDtypeStruct((T, 2), jnp.int32),
            jax.ShapeDtypeStruct((T, 2, 128), jnp.float32),
        ],
        scratch_shapes=[pltpu.VMEM((1, E), jnp.float32)],
        compiler_params=pltpu.CompilerParams(
            dimension_semantics=("arbitrary",)),
        interpret=interpret,
    )(xf, ln_scale.reshape(1, D), ln_bias.reshape(1, D), wg, tril)


NC, NS, L = 2, 16, 16
NW = NC * NS
PAIRS_W = TK // NW
PCH = 64
TOK_W = T // NW
TCH = 16

_sc_mesh = plsc.VectorSubcoreMesh(core_axis_name="c", subcore_axis_name="s")


NCH_D = PAIRS_W // PCH


def _dispatch_body(xn_hbm, dst2_hbm, wb_hbm, disp_hbm, wrow_hbm,
                   dstv, sidx, buf0, buf1, wb0, wb1, gsem, ssem, wsem):
    wid = lax.axis_index("s") * NC + lax.axis_index("c")
    bufs = (buf0, buf1)
    wbs = (wb0, wb1)
    pltpu.sync_copy(dst2_hbm.at[pl.ds(wid * NCH_D, NCH_D)], dstv)
    for g in range(PAIRS_W // L):
        pairv = wid * PAIRS_W + g * L + lax.iota(jnp.int32, L)
        sidx[g // (PCH // L), pl.ds((g % (PCH // L)) * L, L)] = (
            lax.shift_right_logical(pairv, 1))
    pltpu.async_copy(xn_hbm.at[sidx.at[0]], bufs[0], gsem[0])
    for ci in range(NCH_D):
        p = ci % 2
        q = (ci + 1) % 2
        if ci >= 1:
            pltpu.make_async_copy(bufs[q], disp_hbm.at[dstv.at[ci - 1]],
                                  ssem[q]).wait()
            pltpu.make_async_copy(wbs[q], wrow_hbm.at[dstv.at[ci - 1]],
                                  wsem[q]).wait()
        if ci + 1 < NCH_D:
            pltpu.async_copy(xn_hbm.at[sidx.at[ci + 1]], bufs[q], gsem[q])
        pltpu.sync_copy(wb_hbm.at[pl.ds(wid * PAIRS_W + ci * PCH, PCH)],
                        wbs[p])
        pltpu.make_async_copy(xn_hbm.at[sidx.at[ci]], bufs[p], gsem[p]).wait()
        pltpu.async_copy(bufs[p], disp_hbm.at[dstv.at[ci]], ssem[p])
        pltpu.async_copy(wbs[p], wrow_hbm.at[dstv.at[ci]], wsem[p])
    pltpu.make_async_copy(bufs[(NCH_D - 1) % 2],
                          disp_hbm.at[dstv.at[NCH_D - 1]],
                          ssem[(NCH_D - 1) % 2]).wait()
    pltpu.make_async_copy(wbs[(NCH_D - 1) % 2],
                          wrow_hbm.at[dstv.at[NCH_D - 1]],
                          wsem[(NCH_D - 1) % 2]).wait()


def _dispatch(xn, dstf, wb):
    return pl.kernel(
        _dispatch_body,
        out_type=[jax.ShapeDtypeStruct((ECP, D), jnp.float32),
                  jax.ShapeDtypeStruct((ECP, 128), jnp.float32)],
        mesh=_sc_mesh,
        scratch_types=[
            pltpu.VMEM((NCH_D, PCH), jnp.int32),
            pltpu.VMEM((NCH_D, PCH), jnp.int32),
            pltpu.VMEM((PCH, D), jnp.float32),
            pltpu.VMEM((PCH, D), jnp.float32),
            pltpu.VMEM((PCH, 128), jnp.float32),
            pltpu.VMEM((PCH, 128), jnp.float32),
            (pltpu.SemaphoreType.DMA, pltpu.SemaphoreType.DMA),
            (pltpu.SemaphoreType.DMA, pltpu.SemaphoreType.DMA),
            (pltpu.SemaphoreType.DMA, pltpu.SemaphoreType.DMA),
        ],
    )(xn, dstf.reshape(NW * NCH_D, PCH), wb)


NCH_C = TOK_W // TCH


def _combine_body(x_hbm, dst2_hbm, yw_hbm, out_hbm,
                  dstv, xb0, xb1, gb0, gb1, gsem, xsem, osem):
    wid = lax.axis_index("s") * NC + lax.axis_index("c")
    xbs = (xb0, xb1)
    gbs = (gb0, gb1)
    pltpu.sync_copy(dst2_hbm.at[pl.ds(wid * NCH_C, NCH_C)], dstv)
    pltpu.async_copy(yw_hbm.at[dstv.at[0]], gbs[0], gsem[0])
    pltpu.async_copy(x_hbm.at[pl.ds(wid * TOK_W, TCH)], xbs[0], xsem[0])
    for ci in range(NCH_C):
        p = ci % 2
        q = (ci + 1) % 2
        if ci >= 1:
            pltpu.make_async_copy(
                xbs[q], out_hbm.at[pl.ds(wid * TOK_W + (ci - 1) * TCH, TCH)],
                osem[q]).wait()
        if ci + 1 < NCH_C:
            pltpu.async_copy(yw_hbm.at[dstv.at[ci + 1]], gbs[q], gsem[q])
            pltpu.async_copy(
                x_hbm.at[pl.ds(wid * TOK_W + (ci + 1) * TCH, TCH)],
                xbs[q], xsem[q])
        pltpu.make_async_copy(yw_hbm.at[dstv.at[ci]], gbs[p], gsem[p]).wait()
        pltpu.make_async_copy(x_hbm.at[pl.ds(0, TCH)], xbs[p], xsem[p]).wait()

        xb = xbs[p]
        gb = gbs[p]

        def tok_row(t, carry):
            for c in range(D // L):
                sl = pl.ds(c * L, L)
                xb[t, sl] = xb[t, sl] + gb[2 * t, sl] + gb[2 * t + 1, sl]
            return carry

        lax.fori_loop(0, TCH, tok_row, 0)
        pltpu.async_copy(xb, out_hbm.at[pl.ds(wid * TOK_W + ci * TCH, TCH)],
                         osem[p])
    pltpu.make_async_copy(
        xbs[(NCH_C - 1) % 2],
        out_hbm.at[pl.ds(wid * TOK_W + (NCH_C - 1) * TCH, TCH)],
        osem[(NCH_C - 1) % 2]).wait()


def _combine(xf, dstf, yw):
    return pl.kernel(
        _combine_body,
        out_type=jax.ShapeDtypeStruct((T, D), jnp.float32),
        mesh=_sc_mesh,
        scratch_types=[
            pltpu.VMEM((NCH_C, 2 * TCH), jnp.int32),
            pltpu.VMEM((TCH, D), jnp.float32),
            pltpu.VMEM((TCH, D), jnp.float32),
            pltpu.VMEM((2 * TCH, D), jnp.float32),
            pltpu.VMEM((2 * TCH, D), jnp.float32),
            (pltpu.SemaphoreType.DMA, pltpu.SemaphoreType.DMA),
            (pltpu.SemaphoreType.DMA, pltpu.SemaphoreType.DMA),
            (pltpu.SemaphoreType.DMA, pltpu.SemaphoreType.DMA),
        ],
    )(xf, dstf.reshape(NW * NCH_C, 2 * TCH), yw)


CB = CP


def _ffn_body(disp_ref, w1_ref, b1_ref, w2_ref, b2_ref, wr_ref, y_ref):
    xb = disp_ref[...].astype(jnp.bfloat16)
    a = jnp.dot(xb, w1_ref[0].astype(jnp.bfloat16),
                preferred_element_type=jnp.float32) + b1_ref[0]
    hh = 0.5 * a * (1.0 + jax.lax.erf(a * np.float32(1.0 / np.sqrt(2.0))))
    y = jnp.dot(hh.astype(jnp.bfloat16), w2_ref[0].astype(jnp.bfloat16),
                preferred_element_type=jnp.float32) + b2_ref[0]
    y_ref[...] = y * wr_ref[...][:, 0:1]


@functools.partial(jax.jit, static_argnames=("interpret",))
def _ffn(disp, W1, b1, W2, b2, wrow, interpret=False):
    return pl.pallas_call(
        _ffn_body,
        grid=(E,),
        in_specs=[
            pl.BlockSpec((CB, D), lambda e: (e, 0)),
            pl.BlockSpec((1, D, P), lambda e: (e, 0, 0)),
            pl.BlockSpec((1, 1, P), lambda e: (e, 0, 0)),
            pl.BlockSpec((1, P, D), lambda e: (e, 0, 0)),
            pl.BlockSpec((1, 1, D), lambda e: (e, 0, 0)),
            pl.BlockSpec((CB, 128), lambda e: (e, 0)),
        ],
        out_specs=pl.BlockSpec((CB, D), lambda e: (e, 0)),
        out_shape=jax.ShapeDtypeStruct((ECP, D), jnp.float32),
        compiler_params=pltpu.CompilerParams(
            dimension_semantics=("arbitrary",)),
        interpret=interpret,
    )(disp, W1, b1.reshape(E, 1, P), W2, b2.reshape(E, 1, D), wrow)


def kernel(x, ln_scale, ln_bias, Wg, W1, b1, W2, b2):
    xf = x.reshape(T, D)
    xn, dst, wv = _route(xf, ln_scale, ln_bias, Wg)
    dstf = dst.reshape(TK)
    wb = wv.reshape(TK, 128)
    disp, wrow = _dispatch(xn, dstf, wb)
    yw = _ffn(disp, W1, b1, W2, b2, wrow)
    out = _combine(xf, dstf, yw)
    return out.reshape(B, S, D)


# speedup vs baseline: 1.4637x; 1.0893x over previous
import functools

import jax
import jax.numpy as jnp
import numpy as np
from jax import lax
from jax.experimental import pallas as pl
from jax.experimental.pallas import tpu as pltpu
from jax.experimental.pallas import tpu_sc as plsc

B, S, D = 2, 2048, 768
E, K, P = 16, 2, 1536
T = B * S
TK = T * K
C = int(np.ceil(TK * 1.25 / E))
CP = C + 8
ECP = E * CP
TRASH = C
TOK_BLK = 512


def _route_body(x_ref, scale_ref, bias_ref, wg_ref, tril_ref,
                xn_ref, dste_ref, dsto_ref, wv_ref, cnt_ref):
    i = pl.program_id(0)

    @pl.when(i == 0)
    def _():
        cnt_ref[...] = jnp.zeros_like(cnt_ref)

    xb = x_ref[...]
    mu = jnp.mean(xb, axis=-1, keepdims=True)
    xc = xb - mu
    var = jnp.mean(xc * xc, axis=-1, keepdims=True)
    xn = xc * jax.lax.rsqrt(var + 1e-5) * scale_ref[...] + bias_ref[...]
    xn_ref[...] = xn

    logits = jnp.dot(xn, wg_ref[...], preferred_element_type=jnp.float32,
                     precision=jax.lax.Precision.DEFAULT)
    eidx = jax.lax.broadcasted_iota(jnp.int32, (TOK_BLK, E), 1)
    m1 = jnp.max(logits, axis=-1, keepdims=True)
    i1 = jnp.min(jnp.where(logits >= m1, eidx, E), axis=-1, keepdims=True)
    l2 = jnp.where(eidx == i1, -jnp.inf, logits)
    m2 = jnp.max(l2, axis=-1, keepdims=True)
    i2 = jnp.min(jnp.where(l2 >= m2, eidx, E), axis=-1, keepdims=True)
    dexp = jnp.exp(m2 - m1)
    denom = 1.0 + dexp
    w1 = 1.0 / denom
    w2 = dexp / denom

    o1 = (eidx == i1).astype(jnp.float32)
    o2 = (eidx == i2).astype(jnp.float32)
    h = o1 + o2
    s = jnp.dot(tril_ref[...], h, preferred_element_type=jnp.float32) \
        + cnt_ref[...]
    cnt_ref[...] = cnt_ref[...] + jnp.sum(h, axis=0, keepdims=True)
    r1 = jnp.sum(s * o1, axis=-1, keepdims=True)
    r2 = jnp.sum(s * o2, axis=-1, keepdims=True)
    v1 = r1 < C
    v2 = r2 < C
    d1 = jnp.where(v1, i1 * CP + r1.astype(jnp.int32), TRASH)
    d2 = jnp.where(v2, i2 * CP + r2.astype(jnp.int32), TRASH)
    dste_ref[...] = d1
    dsto_ref[...] = d2
    w1m = jnp.where(v1, w1, 0.0)
    w2m = jnp.where(v2, w2, 0.0)
    wv_ref[...] = jnp.concatenate(
        [jnp.broadcast_to(w1m, (TOK_BLK, 128))[None],
         jnp.broadcast_to(w2m, (TOK_BLK, 128))[None]], axis=0)


@functools.partial(jax.jit, static_argnames=("interpret",))
def _route(xf, ln_scale, ln_bias, wg, interpret=False):
    tril = jnp.tril(jnp.ones((TOK_BLK, TOK_BLK), jnp.float32), -1)
    grid = (T // TOK_BLK,)
    return pl.pallas_call(
        _route_body,
        grid=grid,
        in_specs=[
            pl.BlockSpec((TOK_BLK, D), lambda i: (i, 0)),
            pl.BlockSpec((1, D), lambda i: (0, 0)),
            pl.BlockSpec((1, D), lambda i: (0, 0)),
            pl.BlockSpec((D, E), lambda i: (0, 0)),
            pl.BlockSpec((TOK_BLK, TOK_BLK), lambda i: (0, 0)),
        ],
        out_specs=[
            pl.BlockSpec((TOK_BLK, D), lambda i: (i, 0)),
            pl.BlockSpec((TOK_BLK, 1), lambda i: (i, 0)),
            pl.BlockSpec((TOK_BLK, 1), lambda i: (i, 0)),
            pl.BlockSpec((2, TOK_BLK, 128), lambda i: (0, i, 0)),
        ],
        out_shape=[
            jax.ShapeDtypeStruct((T, D), jnp.float32),
            jax.ShapeDtypeStruct((T, 1), jnp.int32),
            jax.ShapeDtypeStruct((T, 1), jnp.int32),
            jax.ShapeDtypeStruct((2, T, 128), jnp.float32),
        ],
        scratch_shapes=[pltpu.VMEM((1, E), jnp.float32)],
        compiler_params=pltpu.CompilerParams(
            dimension_semantics=("arbitrary",)),
        interpret=interpret,
    )(xf, ln_scale.reshape(1, D), ln_bias.reshape(1, D), wg, tril)


NC, NS, L = 2, 16, 16
NW = NC * NS
PAIRS_W = TK // NW
PCH = 64
TOK_W = T // NW
TCH = 16

_sc_mesh = plsc.VectorSubcoreMesh(core_axis_name="c", subcore_axis_name="s")


DT = 32
NCH_D = TOK_W // DT


def _dispatch_body(xn_hbm, dste_hbm, dsto_hbm, wbe_hbm, wbo_hbm,
                   disp_hbm, wrow_hbm,
                   dste, dsto, xb0, xb1, we0, we1, wo0, wo1,
                   lsem, dsem, wsem):
    wid = lax.axis_index("s") * NC + lax.axis_index("c")
    xbs = (xb0, xb1)
    wes = (we0, we1)
    wos = (wo0, wo1)
    tok0 = wid * TOK_W
    pltpu.sync_copy(dste_hbm.at[pl.ds(wid * NCH_D, NCH_D)], dste)
    pltpu.sync_copy(dsto_hbm.at[pl.ds(wid * NCH_D, NCH_D)], dsto)
    pltpu.async_copy(xn_hbm.at[pl.ds(tok0, DT)], xbs[0], lsem[0])
    for ci in range(NCH_D):
        p = ci % 2
        q = (ci + 1) % 2
        if ci >= 1:
            for _ in range(2):
                pltpu.make_async_copy(xbs[q], disp_hbm.at[dste.at[ci - 1]],
                                      dsem[q]).wait()
                pltpu.make_async_copy(wes[q], wrow_hbm.at[dste.at[ci - 1]],
                                      wsem[q]).wait()
        if ci + 1 < NCH_D:
            pltpu.async_copy(xn_hbm.at[pl.ds(tok0 + (ci + 1) * DT, DT)],
                             xbs[q], lsem[q])
        pltpu.sync_copy(wbe_hbm.at[pl.ds(tok0 + ci * DT, DT)], wes[p])
        pltpu.sync_copy(wbo_hbm.at[pl.ds(tok0 + ci * DT, DT)], wos[p])
        pltpu.make_async_copy(xn_hbm.at[pl.ds(tok0, DT)], xbs[p],
                              lsem[p]).wait()
        pltpu.async_copy(xbs[p], disp_hbm.at[dste.at[ci]], dsem[p])
        pltpu.async_copy(xbs[p], disp_hbm.at[dsto.at[ci]], dsem[p])
        pltpu.async_copy(wes[p], wrow_hbm.at[dste.at[ci]], wsem[p])
        pltpu.async_copy(wos[p], wrow_hbm.at[dsto.at[ci]], wsem[p])
    pf = (NCH_D - 1) % 2
    for _ in range(2):
        pltpu.make_async_copy(xbs[pf], disp_hbm.at[dste.at[NCH_D - 1]],
                              dsem[pf]).wait()
        pltpu.make_async_copy(wes[pf], wrow_hbm.at[dste.at[NCH_D - 1]],
                              wsem[pf]).wait()


def _dispatch(xn, dstE, dstO, wv):
    return pl.kernel(
        _dispatch_body,
        out_type=[jax.ShapeDtypeStruct((ECP, D), jnp.float32),
                  jax.ShapeDtypeStruct((ECP, 128), jnp.float32)],
        mesh=_sc_mesh,
        scratch_types=[
            pltpu.VMEM((NCH_D, DT), jnp.int32),
            pltpu.VMEM((NCH_D, DT), jnp.int32),
            pltpu.VMEM((DT, D), jnp.float32),
            pltpu.VMEM((DT, D), jnp.float32),
            pltpu.VMEM((DT, 128), jnp.float32),
            pltpu.VMEM((DT, 128), jnp.float32),
            pltpu.VMEM((DT, 128), jnp.float32),
            pltpu.VMEM((DT, 128), jnp.float32),
            (pltpu.SemaphoreType.DMA, pltpu.SemaphoreType.DMA),
            (pltpu.SemaphoreType.DMA, pltpu.SemaphoreType.DMA),
            (pltpu.SemaphoreType.DMA, pltpu.SemaphoreType.DMA),
        ],
    )(xn, dstE.reshape(NW * NCH_D, DT), dstO.reshape(NW * NCH_D, DT),
      wv[0], wv[1])


NCH_C = TOK_W // TCH


def _combine_body(x_hbm, dste_hbm, dsto_hbm, yw_hbm, out_hbm,
                  dste, dsto, xb0, xb1, ge0, ge1, go0, go1,
                  gesem, gosem, xsem, osem):
    wid = lax.axis_index("s") * NC + lax.axis_index("c")
    xbs = (xb0, xb1)
    ges = (ge0, ge1)
    gos = (go0, go1)
    tok0 = wid * TOK_W
    pltpu.sync_copy(dste_hbm.at[pl.ds(wid * NCH_C, NCH_C)], dste)
    pltpu.sync_copy(dsto_hbm.at[pl.ds(wid * NCH_C, NCH_C)], dsto)
    pltpu.async_copy(yw_hbm.at[dste.at[0]], ges[0], gesem[0])
    pltpu.async_copy(yw_hbm.at[dsto.at[0]], gos[0], gosem[0])
    pltpu.async_copy(x_hbm.at[pl.ds(tok0, TCH)], xbs[0], xsem[0])
    for ci in range(NCH_C):
        p = ci % 2
        q = (ci + 1) % 2
        if ci >= 1:
            pltpu.make_async_copy(
                xbs[q], out_hbm.at[pl.ds(tok0 + (ci - 1) * TCH, TCH)],
                osem[q]).wait()
        if ci + 1 < NCH_C:
            pltpu.async_copy(yw_hbm.at[dste.at[ci + 1]], ges[q], gesem[q])
            pltpu.async_copy(yw_hbm.at[dsto.at[ci + 1]], gos[q], gosem[q])
            pltpu.async_copy(x_hbm.at[pl.ds(tok0 + (ci + 1) * TCH, TCH)],
                             xbs[q], xsem[q])
        pltpu.make_async_copy(yw_hbm.at[dste.at[ci]], ges[p], gesem[p]).wait()
        pltpu.make_async_copy(yw_hbm.at[dsto.at[ci]], gos[p], gosem[p]).wait()
        pltpu.make_async_copy(x_hbm.at[pl.ds(0, TCH)], xbs[p],
                              xsem[p]).wait()

        xb = xbs[p]
        ge = ges[p]
        go = gos[p]

        def tok_row(t, carry):
            for c in range(D // L):
                sl = pl.ds(c * L, L)
                xb[t, sl] = xb[t, sl] + ge[t, sl] + go[t, sl]
            return carry

        lax.fori_loop(0, TCH, tok_row, 0)
        pltpu.async_copy(xb, out_hbm.at[pl.ds(tok0 + ci * TCH, TCH)],
                         osem[p])
    pltpu.make_async_copy(
        xbs[(NCH_C - 1) % 2],
        out_hbm.at[pl.ds(tok0 + (NCH_C - 1) * TCH, TCH)],
        osem[(NCH_C - 1) % 2]).wait()


def _combine(xf, dstE, dstO, yw):
    return pl.kernel(
        _combine_body,
        out_type=jax.ShapeDtypeStruct((T, D), jnp.float32),
        mesh=_sc_mesh,
        scratch_types=[
            pltpu.VMEM((NCH_C, TCH), jnp.int32),
            pltpu.VMEM((NCH_C, TCH), jnp.int32),
            pltpu.VMEM((TCH, D), jnp.float32),
            pltpu.VMEM((TCH, D), jnp.float32),
            pltpu.VMEM((TCH, D), jnp.float32),
            pltpu.VMEM((TCH, D), jnp.float32),
            pltpu.VMEM((TCH, D), jnp.float32),
            pltpu.VMEM((TCH, D), jnp.float32),
            (pltpu.SemaphoreType.DMA, pltpu.SemaphoreType.DMA),
            (pltpu.SemaphoreType.DMA, pltpu.SemaphoreType.DMA),
            (pltpu.SemaphoreType.DMA, pltpu.SemaphoreType.DMA),
            (pltpu.SemaphoreType.DMA, pltpu.SemaphoreType.DMA),
        ],
    )(xf, dstE.reshape(NW * NCH_C, TCH), dstO.reshape(NW * NCH_C, TCH), yw)


CB = CP


def _ffn_body(disp_ref, w1_ref, b1_ref, w2_ref, b2_ref, wr_ref, y_ref):
    xb = disp_ref[...].astype(jnp.bfloat16)
    a = jnp.dot(xb, w1_ref[0].astype(jnp.bfloat16),
                preferred_element_type=jnp.float32) + b1_ref[0]
    hh = 0.5 * a * (1.0 + jax.lax.erf(a * np.float32(1.0 / np.sqrt(2.0))))
    y = jnp.dot(hh.astype(jnp.bfloat16), w2_ref[0].astype(jnp.bfloat16),
                preferred_element_type=jnp.float32) + b2_ref[0]
    y_ref[...] = y * wr_ref[...][:, 0:1]


@functools.partial(jax.jit, static_argnames=("interpret",))
def _ffn(disp, W1, b1, W2, b2, wrow, interpret=False):
    return pl.pallas_call(
        _ffn_body,
        grid=(E,),
        in_specs=[
            pl.BlockSpec((CB, D), lambda e: (e, 0)),
            pl.BlockSpec((1, D, P), lambda e: (e, 0, 0)),
            pl.BlockSpec((1, 1, P), lambda e: (e, 0, 0)),
            pl.BlockSpec((1, P, D), lambda e: (e, 0, 0)),
            pl.BlockSpec((1, 1, D), lambda e: (e, 0, 0)),
            pl.BlockSpec((CB, 128), lambda e: (e, 0)),
        ],
        out_specs=pl.BlockSpec((CB, D), lambda e: (e, 0)),
        out_shape=jax.ShapeDtypeStruct((ECP, D), jnp.float32),
        compiler_params=pltpu.CompilerParams(
            dimension_semantics=("arbitrary",)),
        interpret=interpret,
    )(disp, W1, b1.reshape(E, 1, P), W2, b2.reshape(E, 1, D), wrow)


def kernel(x, ln_scale, ln_bias, Wg, W1, b1, W2, b2):
    xf = x.reshape(T, D)
    xn, dstE, dstO, wv = _route(xf, ln_scale, ln_bias, Wg)
    disp, wrow = _dispatch(xn, dstE, dstO, wv)
    yw = _ffn(disp, W1, b1, W2, b2, wrow)
    out = _combine(xf, dstE, dstO, yw)
    return out.reshape(B, S, D)


# speedup vs baseline: 1.4650x; 1.0009x over previous
import functools

import jax
import jax.numpy as jnp
import numpy as np
from jax import lax
from jax.experimental import pallas as pl
from jax.experimental.pallas import tpu as pltpu
from jax.experimental.pallas import tpu_sc as plsc

B, S, D = 2, 2048, 768
E, K, P = 16, 2, 1536
T = B * S
TK = T * K
C = int(np.ceil(TK * 1.25 / E))
CP = C + 8
ECP = E * CP
TRASH = C
TOK_BLK = 512


def _route_body(x_ref, scale_ref, bias_ref, wg_ref, tril_ref,
                xn_ref, dste_ref, dsto_ref, wv_ref, cnt_ref):
    i = pl.program_id(0)

    @pl.when(i == 0)
    def _():
        cnt_ref[...] = jnp.zeros_like(cnt_ref)

    xb = x_ref[...]
    mu = jnp.mean(xb, axis=-1, keepdims=True)
    xc = xb - mu
    var = jnp.mean(xc * xc, axis=-1, keepdims=True)
    xn = xc * jax.lax.rsqrt(var + 1e-5) * scale_ref[...] + bias_ref[...]
    xn_ref[...] = xn

    logits = jnp.dot(xn, wg_ref[...], preferred_element_type=jnp.float32,
                     precision=jax.lax.Precision.DEFAULT)
    eidx = jax.lax.broadcasted_iota(jnp.int32, (TOK_BLK, E), 1)
    m1 = jnp.max(logits, axis=-1, keepdims=True)
    i1 = jnp.min(jnp.where(logits >= m1, eidx, E), axis=-1, keepdims=True)
    l2 = jnp.where(eidx == i1, -jnp.inf, logits)
    m2 = jnp.max(l2, axis=-1, keepdims=True)
    i2 = jnp.min(jnp.where(l2 >= m2, eidx, E), axis=-1, keepdims=True)
    dexp = jnp.exp(m2 - m1)
    denom = 1.0 + dexp
    w1 = 1.0 / denom
    w2 = dexp / denom

    o1 = (eidx == i1).astype(jnp.float32)
    o2 = (eidx == i2).astype(jnp.float32)
    h = o1 + o2
    s = jnp.dot(tril_ref[...], h, preferred_element_type=jnp.float32) \
        + cnt_ref[...]
    cnt_ref[...] = cnt_ref[...] + jnp.sum(h, axis=0, keepdims=True)
    r1 = jnp.sum(s * o1, axis=-1, keepdims=True)
    r2 = jnp.sum(s * o2, axis=-1, keepdims=True)
    v1 = r1 < C
    v2 = r2 < C
    d1 = jnp.where(v1, i1 * CP + r1.astype(jnp.int32), TRASH)
    d2 = jnp.where(v2, i2 * CP + r2.astype(jnp.int32), TRASH)
    dste_ref[...] = d1
    dsto_ref[...] = d2
    w1m = jnp.where(v1, w1, 0.0)
    w2m = jnp.where(v2, w2, 0.0)
    wv_ref[...] = jnp.concatenate(
        [jnp.broadcast_to(w1m, (TOK_BLK, 128))[None],
         jnp.broadcast_to(w2m, (TOK_BLK, 128))[None]], axis=0)


@functools.partial(jax.jit, static_argnames=("interpret",))
def _route(xf, ln_scale, ln_bias, wg, interpret=False):
    tril = jnp.tril(jnp.ones((TOK_BLK, TOK_BLK), jnp.float32), -1)
    grid = (T // TOK_BLK,)
    return pl.pallas_call(
        _route_body,
        grid=grid,
        in_specs=[
            pl.BlockSpec((TOK_BLK, D), lambda i: (i, 0)),
            pl.BlockSpec((1, D), lambda i: (0, 0)),
            pl.BlockSpec((1, D), lambda i: (0, 0)),
            pl.BlockSpec((D, E), lambda i: (0, 0)),
            pl.BlockSpec((TOK_BLK, TOK_BLK), lambda i: (0, 0)),
        ],
        out_specs=[
            pl.BlockSpec((TOK_BLK, D), lambda i: (i, 0)),
            pl.BlockSpec((TOK_BLK, 1), lambda i: (i, 0)),
            pl.BlockSpec((TOK_BLK, 1), lambda i: (i, 0)),
            pl.BlockSpec((2, TOK_BLK, 128), lambda i: (0, i, 0)),
        ],
        out_shape=[
            jax.ShapeDtypeStruct((T, D), jnp.float32),
            jax.ShapeDtypeStruct((T, 1), jnp.int32),
            jax.ShapeDtypeStruct((T, 1), jnp.int32),
            jax.ShapeDtypeStruct((2, T, 128), jnp.float32),
        ],
        scratch_shapes=[pltpu.VMEM((1, E), jnp.float32)],
        compiler_params=pltpu.CompilerParams(
            dimension_semantics=("arbitrary",)),
        interpret=interpret,
    )(xf, ln_scale.reshape(1, D), ln_bias.reshape(1, D), wg, tril)


NC, NS, L = 2, 16, 16
NW = NC * NS
TOK_W = T // NW
TCH = 16

_sc_mesh = plsc.VectorSubcoreMesh(core_axis_name="c", subcore_axis_name="s")


DT = 32
NCH_D = TOK_W // DT


def _dispatch_body(xn_hbm, dste_hbm, dsto_hbm, wbe_hbm, wbo_hbm,
                   disp_hbm, wrow_hbm,
                   dste, dsto, xb0, xb1, we0, we1, wo0, wo1,
                   lsem, dsem, wsem):
    wid = lax.axis_index("s") * NC + lax.axis_index("c")
    xbs = (xb0, xb1)
    wes = (we0, we1)
    wos = (wo0, wo1)
    tok0 = wid * TOK_W
    pltpu.sync_copy(dste_hbm.at[pl.ds(wid * NCH_D, NCH_D)], dste)
    pltpu.sync_copy(dsto_hbm.at[pl.ds(wid * NCH_D, NCH_D)], dsto)
    pltpu.async_copy(xn_hbm.at[pl.ds(tok0, DT)], xbs[0], lsem[0])
    for ci in range(NCH_D):
        p = ci % 2
        q = (ci + 1) % 2
        if ci >= 1:
            for _ in range(2):
                pltpu.make_async_copy(xbs[q], disp_hbm.at[dste.at[ci - 1]],
                                      dsem[q]).wait()
                pltpu.make_async_copy(wes[q], wrow_hbm.at[dste.at[ci - 1]],
                                      wsem[q]).wait()
        if ci + 1 < NCH_D:
            pltpu.async_copy(xn_hbm.at[pl.ds(tok0 + (ci + 1) * DT, DT)],
                             xbs[q], lsem[q])
        pltpu.sync_copy(wbe_hbm.at[pl.ds(tok0 + ci * DT, DT)], wes[p])
        pltpu.sync_copy(wbo_hbm.at[pl.ds(tok0 + ci * DT, DT)], wos[p])
        pltpu.make_async_copy(xn_hbm.at[pl.ds(tok0, DT)], xbs[p],
                              lsem[p]).wait()
        pltpu.async_copy(xbs[p], disp_hbm.at[dste.at[ci]], dsem[p])
        pltpu.async_copy(xbs[p], disp_hbm.at[dsto.at[ci]], dsem[p])
        pltpu.async_copy(wes[p], wrow_hbm.at[dste.at[ci]], wsem[p])
        pltpu.async_copy(wos[p], wrow_hbm.at[dsto.at[ci]], wsem[p])
    pf = (NCH_D - 1) % 2
    for _ in range(2):
        pltpu.make_async_copy(xbs[pf], disp_hbm.at[dste.at[NCH_D - 1]],
                              dsem[pf]).wait()
        pltpu.make_async_copy(wes[pf], wrow_hbm.at[dste.at[NCH_D - 1]],
                              wsem[pf]).wait()


def _dispatch(xn, dstE, dstO, wv):
    return pl.kernel(
        _dispatch_body,
        out_type=[jax.ShapeDtypeStruct((ECP, D), jnp.float32),
                  jax.ShapeDtypeStruct((ECP, 128), jnp.float32)],
        mesh=_sc_mesh,
        scratch_types=[
            pltpu.VMEM((NCH_D, DT), jnp.int32),
            pltpu.VMEM((NCH_D, DT), jnp.int32),
            pltpu.VMEM((DT, D), jnp.float32),
            pltpu.VMEM((DT, D), jnp.float32),
            pltpu.VMEM((DT, 128), jnp.float32),
            pltpu.VMEM((DT, 128), jnp.float32),
            pltpu.VMEM((DT, 128), jnp.float32),
            pltpu.VMEM((DT, 128), jnp.float32),
            (pltpu.SemaphoreType.DMA, pltpu.SemaphoreType.DMA),
            (pltpu.SemaphoreType.DMA, pltpu.SemaphoreType.DMA),
            (pltpu.SemaphoreType.DMA, pltpu.SemaphoreType.DMA),
        ],
    )(xn, dstE.reshape(NW * NCH_D, DT), dstO.reshape(NW * NCH_D, DT),
      wv[0], wv[1])


NCH_C = TOK_W // TCH


def _combine_body(x_hbm, dste_hbm, dsto_hbm, yw_hbm, out_hbm,
                  dste, dsto, xb0, xb1, ge0, ge1, go0, go1,
                  gesem, gosem, xsem, osem):
    wid = lax.axis_index("s") * NC + lax.axis_index("c")
    xbs = (xb0, xb1)
    ges = (ge0, ge1)
    gos = (go0, go1)
    tok0 = wid * TOK_W
    pltpu.sync_copy(dste_hbm.at[pl.ds(wid * NCH_C, NCH_C)], dste)
    pltpu.sync_copy(dsto_hbm.at[pl.ds(wid * NCH_C, NCH_C)], dsto)
    pltpu.async_copy(yw_hbm.at[dste.at[0]], ges[0], gesem[0])
    pltpu.async_copy(yw_hbm.at[dsto.at[0]], gos[0], gosem[0])
    pltpu.async_copy(x_hbm.at[pl.ds(tok0, TCH)], xbs[0], xsem[0])
    for ci in range(NCH_C):
        p = ci % 2
        q = (ci + 1) % 2
        if ci >= 1:
            pltpu.make_async_copy(
                xbs[q], out_hbm.at[pl.ds(tok0 + (ci - 1) * TCH, TCH)],
                osem[q]).wait()
        if ci + 1 < NCH_C:
            pltpu.async_copy(yw_hbm.at[dste.at[ci + 1]], ges[q], gesem[q])
            pltpu.async_copy(yw_hbm.at[dsto.at[ci + 1]], gos[q], gosem[q])
            pltpu.async_copy(x_hbm.at[pl.ds(tok0 + (ci + 1) * TCH, TCH)],
                             xbs[q], xsem[q])
        pltpu.make_async_copy(yw_hbm.at[dste.at[ci]], ges[p], gesem[p]).wait()
        pltpu.make_async_copy(yw_hbm.at[dsto.at[ci]], gos[p], gosem[p]).wait()
        pltpu.make_async_copy(x_hbm.at[pl.ds(0, TCH)], xbs[p],
                              xsem[p]).wait()

        xb = xbs[p]
        ge = ges[p]
        go = gos[p]

        def tok_row(t, carry):
            for c in range(D // L):
                sl = pl.ds(c * L, L)
                xb[t, sl] = xb[t, sl] + ge[t, sl] + go[t, sl]
            return carry

        lax.fori_loop(0, TCH, tok_row, 0)
        pltpu.async_copy(xb, out_hbm.at[pl.ds(tok0 + ci * TCH, TCH)],
                         osem[p])
    pltpu.make_async_copy(
        xbs[(NCH_C - 1) % 2],
        out_hbm.at[pl.ds(tok0 + (NCH_C - 1) * TCH, TCH)],
        osem[(NCH_C - 1) % 2]).wait()


def _combine(xf, dstE, dstO, yw):
    return pl.kernel(
        _combine_body,
        out_type=jax.ShapeDtypeStruct((T, D), jnp.float32),
        mesh=_sc_mesh,
        scratch_types=[
            pltpu.VMEM((NCH_C, TCH), jnp.int32),
            pltpu.VMEM((NCH_C, TCH), jnp.int32),
            pltpu.VMEM((TCH, D), jnp.float32),
            pltpu.VMEM((TCH, D), jnp.float32),
            pltpu.VMEM((TCH, D), jnp.float32),
            pltpu.VMEM((TCH, D), jnp.float32),
            pltpu.VMEM((TCH, D), jnp.float32),
            pltpu.VMEM((TCH, D), jnp.float32),
            (pltpu.SemaphoreType.DMA, pltpu.SemaphoreType.DMA),
            (pltpu.SemaphoreType.DMA, pltpu.SemaphoreType.DMA),
            (pltpu.SemaphoreType.DMA, pltpu.SemaphoreType.DMA),
            (pltpu.SemaphoreType.DMA, pltpu.SemaphoreType.DMA),
        ],
    )(xf, dstE.reshape(NW * NCH_C, TCH), dstO.reshape(NW * NCH_C, TCH), yw)


CB = CP


def _ffn_body(disp_ref, w1_ref, b1_ref, w2_ref, b2_ref, wr_ref, y_ref):
    xb = disp_ref[...].astype(jnp.bfloat16)
    a = jnp.dot(xb, w1_ref[0].astype(jnp.bfloat16),
                preferred_element_type=jnp.float32) + b1_ref[0]
    hh = 0.5 * a * (1.0 + jax.lax.erf(a * np.float32(1.0 / np.sqrt(2.0))))
    y = jnp.dot(hh.astype(jnp.bfloat16), w2_ref[0].astype(jnp.bfloat16),
                preferred_element_type=jnp.float32) + b2_ref[0]
    y_ref[...] = y * wr_ref[...][:, 0:1]


@functools.partial(jax.jit, static_argnames=("interpret",))
def _ffn(disp, W1, b1, W2, b2, wrow, interpret=False):
    return pl.pallas_call(
        _ffn_body,
        grid=(E,),
        in_specs=[
            pl.BlockSpec((CB, D), lambda e: (e, 0)),
            pl.BlockSpec((1, D, P), lambda e: (e, 0, 0)),
            pl.BlockSpec((1, 1, P), lambda e: (e, 0, 0)),
            pl.BlockSpec((1, P, D), lambda e: (e, 0, 0)),
            pl.BlockSpec((1, 1, D), lambda e: (e, 0, 0)),
            pl.BlockSpec((CB, 128), lambda e: (e, 0)),
        ],
        out_specs=pl.BlockSpec((CB, D), lambda e: (e, 0)),
        out_shape=jax.ShapeDtypeStruct((ECP, D), jnp.float32),
        compiler_params=pltpu.CompilerParams(
            dimension_semantics=("arbitrary",)),
        interpret=interpret,
    )(disp, W1, b1.reshape(E, 1, P), W2, b2.reshape(E, 1, D), wrow)


def kernel(x, ln_scale, ln_bias, Wg, W1, b1, W2, b2):
    xf = x.reshape(T, D)
    xn, dstE, dstO, wv = _route(xf, ln_scale, ln_bias, Wg)
    disp, wrow = _dispatch(xn, dstE, dstO, wv)
    yw = _ffn(disp, W1, b1, W2, b2, wrow)
    out = _combine(xf, dstE, dstO, yw)
    return out.reshape(B, S, D)
